# baseline (speedup 1.0000x reference)
import jax
import jax.numpy as jnp
from jax.experimental import pallas as pl
from jax.experimental.pallas import tpu as pltpu

N = 128
F = 64
H = 64
B = 32
T = 60
NMAT = 5
CH = 32
C_SEIZ = 4
C_ID = 100


def _dot(a, b):
    return jnp.dot(a, b, preferred_element_type=jnp.float32)


def _bf(x):
    return x.astype(jnp.bfloat16)


def _dcrnn_body(seq_ref, idx_ref, sup_ref,
                wg0_ref, bg0_ref, wc0_ref, bc0_ref,
                wg1_ref, bg1_ref, wc1_ref, bc1_ref,
                wfc_ref, bfc_ref, wid_ref, bid_ref,
                out_seiz_ref, out_id_ref,
                st1, st2, last):
    t = pl.program_id(0)

    @pl.when(t == 0)
    def _init():
        st1[:] = jnp.zeros((N, B * H), jnp.float32)
        st2[:] = jnp.zeros((N, B * H), jnp.float32)
        last[:] = jnp.zeros((B, N, H), jnp.float32)

    s0 = sup_ref[0]
    s1 = sup_ref[1]
    s0d = sup_ref[2]
    s1d = sup_ref[3]

    def diffuse(xb):
        xf = xb.astype(jnp.float32)
        a1 = _bf(_dot(s0, xb))
        a2 = _bf(_dot(s0d, a1) - xf)
        e1 = _bf(_dot(s1, xb))
        e2 = _bf(_dot(s1d, e1) - xf)
        return [xb, a1, a2, e1, e2]

    def gru_step(x, st_ref, wg_ref, bg_ref, wc_ref, bc_ref):
        st = st_ref[:]
        dx = diffuse(x)
        dst = diffuse(_bf(st))
        us = []
        rs = []
        for b in range(B):
            parts = []
            for m in range(NMAT):
                parts.append(dx[m][:, b * F:(b + 1) * F])
                parts.append(dst[m][:, b * H:(b + 1) * H])
            xcat = jnp.concatenate(parts, axis=1)
            g = jax.nn.sigmoid(_dot(xcat, wg_ref[:]) + bg_ref[:])
            rs.append(g[:, :H] * st[:, b * H:(b + 1) * H])
            us.append(g[:, H:])
        drst = diffuse(_bf(jnp.concatenate(rs, axis=1)))
        outs = []
        for b in range(B):
            parts = []
            for m in range(NMAT):
                parts.append(dx[m][:, b * F:(b + 1) * F])
                parts.append(drst[m][:, b * H:(b + 1) * H])
            ccat = jnp.concatenate(parts, axis=1)
            cnd = jnp.tanh(_dot(ccat, wc_ref[:]) + bc_ref[:])
            st_b = st[:, b * H:(b + 1) * H]
            outs.append(us[b] * st_b + (1.0 - us[b]) * cnd)
        for b in range(B):
            st_ref[:, b * H:(b + 1) * H] = outs[b]
        return outs

    mid = gru_step(seq_ref[0], st1, wg0_ref, bg0_ref, wc0_ref, bc0_ref)
    x2 = _bf(jnp.concatenate(mid, axis=1))
    fin = gru_step(x2, st2, wg1_ref, bg1_ref, wc1_ref, bc1_ref)
    for b in range(B):
        @pl.when(idx_ref[b] == t)
        def _store(b=b):
            last[b] = fin[b]

    @pl.when(t == T - 1)
    def _heads():
        h = jnp.maximum(last[:], 0.0).reshape(B * N, H)
        lg = _dot(h, wfc_ref[:]) + bfc_ref[:]
        out_seiz_ref[:] = jnp.max(lg.reshape(B, N, C_SEIZ), axis=1)
        li = _dot(h, wid_ref[:]) + bid_ref[:]
        out_id_ref[:] = jnp.max(li.reshape(B, N, C_ID), axis=1)


def kernel(input_seq, seq_lengths, supports, W_g0, b_g0, W_c0, b_c0,
           W_g1, b_g1, W_c1, b_c1, W_fc, b_fc, W_id, b_id):
    seq = jnp.transpose(input_seq, (1, 2, 0, 3)).reshape(T, N, B * F)
    seq = seq.astype(jnp.bfloat16)
    idx = jnp.clip(seq_lengths.astype(jnp.int32) - 1, 0, T - 1)
    sups = jnp.concatenate([supports, 2.0 * supports], axis=0)
    sups = sups.astype(jnp.bfloat16)

    def regroup(w, in_size):
        return w.reshape(in_size, NMAT, -1).transpose(1, 0, 2).reshape(
            NMAT * in_size, -1).astype(jnp.bfloat16)

    wg0 = regroup(W_g0, F + H)
    wc0 = regroup(W_c0, F + H)
    wg1 = regroup(W_g1, F + H)
    wc1 = regroup(W_c1, F + H)

    full = lambda a: pl.BlockSpec(a.shape, lambda t: (0,) * a.ndim)
    args = (seq, idx, sups,
            wg0, b_g0.reshape(1, -1), wc0, b_c0.reshape(1, -1),
            wg1, b_g1.reshape(1, -1), wc1, b_c1.reshape(1, -1),
            W_fc, b_fc.reshape(1, -1), W_id, b_id.reshape(1, -1))
    in_specs = [
        pl.BlockSpec((1, N, B * F), lambda t: (t, 0, 0)),
        pl.BlockSpec(memory_space=pltpu.SMEM),
    ] + [full(a) for a in args[2:]]

    out_seiz, out_id = pl.pallas_call(
        _dcrnn_body,
        grid=(T,),
        in_specs=in_specs,
        out_specs=[
            pl.BlockSpec((B, C_SEIZ), lambda t: (0, 0)),
            pl.BlockSpec((B, C_ID), lambda t: (0, 0)),
        ],
        out_shape=[
            jax.ShapeDtypeStruct((B, C_SEIZ), jnp.float32),
            jax.ShapeDtypeStruct((B, C_ID), jnp.float32),
        ],
        scratch_shapes=[
            pltpu.VMEM((N, B * H), jnp.float32),
            pltpu.VMEM((N, B * H), jnp.float32),
            pltpu.VMEM((B, N, H), jnp.float32),
        ],
        compiler_params=pltpu.CompilerParams(
            dimension_semantics=("arbitrary",),
        ),
    )(*args)
    return (out_seiz, out_id)


# speedup vs baseline: 1.1017x; 1.1017x over previous
import jax
import jax.numpy as jnp
from jax.experimental import pallas as pl
from jax.experimental.pallas import tpu as pltpu

N = 128
F = 64
H = 64
B = 32
T = 60
NMAT = 5
CH = 32
C_SEIZ = 4
C_ID = 100


def _dot(a, b):
    return jnp.dot(a, b, preferred_element_type=jnp.float32)


def _bf(x):
    return x.astype(jnp.bfloat16)


def _dcrnn_body(seq_ref, idx_ref, supA_ref, supB_ref,
                wg0_ref, bg0_ref, wc0_ref, bc0_ref,
                wg1_ref, bg1_ref, wc1_ref, bc1_ref,
                wfc_ref, bfc_ref, wid_ref, bid_ref,
                out_seiz_ref, out_id_ref,
                st1, st2, mid_s, last):
    t = pl.program_id(0)

    @pl.when(t == 0)
    def _init():
        st1[:] = jnp.zeros((N, B * H), jnp.float32)
        st2[:] = jnp.zeros((N, B * H), jnp.float32)
        mid_s[:] = jnp.zeros((N, B * H), jnp.bfloat16)
        last[:] = jnp.zeros((B, N, H), jnp.float32)

    supA = supA_ref[:]
    supB = supB_ref[:]

    def diffuse(xb):
        xf = xb.astype(jnp.float32)
        d1 = _bf(_dot(supA, xb))
        d2 = _dot(supB, d1)
        a2 = _bf(d2[:N] - xf)
        e2 = _bf(d2[N:] - xf)
        return [xb, d1[:N], a2, d1[N:], e2]

    def gru_step(x, st_ref, wg_ref, bg_ref, wc_ref, bc_ref, live=None):
        st = st_ref[:]
        dx = diffuse(x)
        dst = diffuse(_bf(st))
        def cat(d5, b):
            parts = []
            for m in range(NMAT):
                parts.append(dx[m][:, b * F:(b + 1) * F])
                parts.append(d5[m][:, b * H:(b + 1) * H])
            return jnp.concatenate(parts, axis=1)

        us = []
        rs = []
        for p in range(B // 2):
            b = 2 * p
            xcat2 = jnp.concatenate([cat(dst, b), cat(dst, b + 1)], axis=0)
            g2 = jax.nn.sigmoid(_dot(xcat2, wg_ref[:]) + bg_ref[:])
            for k in range(2):
                g = g2[k * N:(k + 1) * N]
                rs.append(g[:, :H] * st[:, (b + k) * H:(b + k + 1) * H])
                us.append(g[:, H:])
        drst = diffuse(_bf(jnp.concatenate(rs, axis=1)))
        outs = []
        for p in range(B // 2):
            b = 2 * p
            ccat2 = jnp.concatenate([cat(drst, b), cat(drst, b + 1)], axis=0)
            c2 = jnp.tanh(_dot(ccat2, wc_ref[:]) + bc_ref[:])
            for k in range(2):
                cnd = c2[k * N:(k + 1) * N]
                st_b = st[:, (b + k) * H:(b + k + 1) * H]
                outs.append(us[b + k] * st_b + (1.0 - us[b + k]) * cnd)
        if live is not None:
            outs = [jnp.where(live, o, 0.0) for o in outs]
        for b in range(B):
            st_ref[:, b * H:(b + 1) * H] = outs[b]
        return outs

    x2 = mid_s[:]
    fin = gru_step(x2, st2, wg1_ref, bg1_ref, wc1_ref, bc1_ref, live=t > 0)
    live2 = t > 0
    mid = gru_step(seq_ref[0], st1, wg0_ref, bg0_ref, wc0_ref, bc0_ref)
    for b in range(B):
        mid_s[:, b * H:(b + 1) * H] = _bf(mid[b])
    for b in range(B):
        @pl.when(jnp.logical_and(idx_ref[b] == t - 1, live2))
        def _store(b=b):
            last[b] = fin[b]

    @pl.when(t == T)
    def _heads():
        h = jnp.maximum(last[:], 0.0).reshape(B * N, H)
        lg = _dot(h, wfc_ref[:]) + bfc_ref[:]
        out_seiz_ref[:] = jnp.max(lg.reshape(B, N, C_SEIZ), axis=1)
        li = _dot(h, wid_ref[:]) + bid_ref[:]
        out_id_ref[:] = jnp.max(li.reshape(B, N, C_ID), axis=1)


def kernel(input_seq, seq_lengths, supports, W_g0, b_g0, W_c0, b_c0,
           W_g1, b_g1, W_c1, b_c1, W_fc, b_fc, W_id, b_id):
    seq = jnp.transpose(input_seq, (1, 2, 0, 3)).reshape(T, N, B * F)
    seq = seq.astype(jnp.bfloat16)
    idx = jnp.clip(seq_lengths.astype(jnp.int32) - 1, 0, T - 1)
    supA = supports.reshape(2 * N, N).astype(jnp.bfloat16)
    z = jnp.zeros((N, N), supports.dtype)
    supB = jnp.block([[2.0 * supports[0], z], [z, 2.0 * supports[1]]])
    supB = supB.astype(jnp.bfloat16)

    def regroup(w, in_size):
        return w.reshape(in_size, NMAT, -1).transpose(1, 0, 2).reshape(
            NMAT * in_size, -1).astype(jnp.bfloat16)

    wg0 = regroup(W_g0, F + H)
    wc0 = regroup(W_c0, F + H)
    wg1 = regroup(W_g1, F + H)
    wc1 = regroup(W_c1, F + H)

    full = lambda a: pl.BlockSpec(a.shape, lambda t: (0,) * a.ndim)
    args = (seq, idx, supA, supB,
            wg0, b_g0.reshape(1, -1), wc0, b_c0.reshape(1, -1),
            wg1, b_g1.reshape(1, -1), wc1, b_c1.reshape(1, -1),
            W_fc, b_fc.reshape(1, -1), W_id, b_id.reshape(1, -1))
    in_specs = [
        pl.BlockSpec((1, N, B * F), lambda t: (jnp.minimum(t, T - 1), 0, 0)),
        pl.BlockSpec(memory_space=pltpu.SMEM),
    ] + [full(a) for a in args[2:]]

    out_seiz, out_id = pl.pallas_call(
        _dcrnn_body,
        grid=(T + 1,),
        in_specs=in_specs,
        out_specs=[
            pl.BlockSpec((B, C_SEIZ), lambda t: (0, 0)),
            pl.BlockSpec((B, C_ID), lambda t: (0, 0)),
        ],
        out_shape=[
            jax.ShapeDtypeStruct((B, C_SEIZ), jnp.float32),
            jax.ShapeDtypeStruct((B, C_ID), jnp.float32),
        ],
        scratch_shapes=[
            pltpu.VMEM((N, B * H), jnp.float32),
            pltpu.VMEM((N, B * H), jnp.float32),
            pltpu.VMEM((N, B * H), jnp.bfloat16),
            pltpu.VMEM((B, N, H), jnp.float32),
        ],
        compiler_params=pltpu.CompilerParams(
            dimension_semantics=("arbitrary",),
        ),
    )(*args)
    return (out_seiz, out_id)


# speedup vs baseline: 1.1307x; 1.0263x over previous
import jax
import jax.numpy as jnp
from jax.experimental import pallas as pl
from jax.experimental.pallas import tpu as pltpu

N = 128
F = 64
H = 64
B = 32
T = 60
NMAT = 5
CH = 32
C_SEIZ = 4
C_ID = 100


def _dot(a, b):
    return jnp.dot(a, b, preferred_element_type=jnp.float32)


def _bf(x):
    return x.astype(jnp.bfloat16)


def _dcrnn_body(seq_ref, idx_ref, supA_ref, supB_ref,
                wg0_ref, bg0_ref, wc0_ref, bc0_ref,
                wg1_ref, bg1_ref, wc1_ref, bc1_ref,
                wfc_ref, bfc_ref, wid_ref, bid_ref,
                out_seiz_ref, out_id_ref,
                st1, st2, mid_s, last):
    t = pl.program_id(0)

    @pl.when(t == 0)
    def _init():
        st1[:] = jnp.zeros((N, B * H), jnp.float32)
        st2[:] = jnp.zeros((N, B * H), jnp.float32)
        mid_s[:] = jnp.zeros((N, B * H), jnp.bfloat16)
        last[:] = jnp.zeros((B, N, H), jnp.float32)

    supA = supA_ref[:]
    supB = supB_ref[:]

    def diffuse(xb):
        xf = xb.astype(jnp.float32)
        d1 = _bf(_dot(supA, xb))
        d2 = _dot(supB, d1)
        a2 = _bf(d2[:N] - xf)
        e2 = _bf(d2[N:] - xf)
        return [xb, d1[:N], a2, d1[N:], e2]

    def gru_step(x, st_ref, wg_ref, bg_ref, wc_ref, bc_ref, live=None):
        st = st_ref[:]
        dx = diffuse(x)
        dst = diffuse(_bf(st))
        def cat(d5, b):
            parts = []
            for m in range(NMAT):
                parts.append(dx[m][:, b * F:(b + 1) * F])
                parts.append(d5[m][:, b * H:(b + 1) * H])
            return jnp.concatenate(parts, axis=1)

        GRP = 8
        us = []
        rs = []
        for p in range(B // GRP):
            b = GRP * p
            xcat2 = jnp.concatenate([cat(dst, b + k) for k in range(GRP)],
                                    axis=0)
            g2 = jax.nn.sigmoid(_dot(xcat2, wg_ref[:]) + bg_ref[:])
            for k in range(GRP):
                g = g2[k * N:(k + 1) * N]
                rs.append(g[:, :H] * st[:, (b + k) * H:(b + k + 1) * H])
                us.append(g[:, H:])
        drst = diffuse(_bf(jnp.concatenate(rs, axis=1)))
        outs = []
        for p in range(B // GRP):
            b = GRP * p
            ccat2 = jnp.concatenate([cat(drst, b + k) for k in range(GRP)],
                                    axis=0)
            c2 = jnp.tanh(_dot(ccat2, wc_ref[:]) + bc_ref[:])
            for k in range(GRP):
                cnd = c2[k * N:(k + 1) * N]
                st_b = st[:, (b + k) * H:(b + k + 1) * H]
                outs.append(us[b + k] * st_b + (1.0 - us[b + k]) * cnd)
        if live is not None:
            outs = [jnp.where(live, o, 0.0) for o in outs]
        for b in range(B):
            st_ref[:, b * H:(b + 1) * H] = outs[b]
        return outs

    x2 = mid_s[:]
    fin = gru_step(x2, st2, wg1_ref, bg1_ref, wc1_ref, bc1_ref, live=t > 0)
    live2 = t > 0
    mid = gru_step(seq_ref[0], st1, wg0_ref, bg0_ref, wc0_ref, bc0_ref)
    for b in range(B):
        mid_s[:, b * H:(b + 1) * H] = _bf(mid[b])
    for b in range(B):
        @pl.when(jnp.logical_and(idx_ref[b] == t - 1, live2))
        def _store(b=b):
            last[b] = fin[b]

    @pl.when(t == T)
    def _heads():
        h = jnp.maximum(last[:], 0.0).reshape(B * N, H)
        lg = _dot(h, wfc_ref[:]) + bfc_ref[:]
        out_seiz_ref[:] = jnp.max(lg.reshape(B, N, C_SEIZ), axis=1)
        li = _dot(h, wid_ref[:]) + bid_ref[:]
        out_id_ref[:] = jnp.max(li.reshape(B, N, C_ID), axis=1)


def kernel(input_seq, seq_lengths, supports, W_g0, b_g0, W_c0, b_c0,
           W_g1, b_g1, W_c1, b_c1, W_fc, b_fc, W_id, b_id):
    seq = jnp.transpose(input_seq, (1, 2, 0, 3)).reshape(T, N, B * F)
    seq = seq.astype(jnp.bfloat16)
    idx = jnp.clip(seq_lengths.astype(jnp.int32) - 1, 0, T - 1)
    supA = supports.reshape(2 * N, N).astype(jnp.bfloat16)
    z = jnp.zeros((N, N), supports.dtype)
    supB = jnp.block([[2.0 * supports[0], z], [z, 2.0 * supports[1]]])
    supB = supB.astype(jnp.bfloat16)

    def regroup(w, in_size):
        return w.reshape(in_size, NMAT, -1).transpose(1, 0, 2).reshape(
            NMAT * in_size, -1).astype(jnp.bfloat16)

    wg0 = regroup(W_g0, F + H)
    wc0 = regroup(W_c0, F + H)
    wg1 = regroup(W_g1, F + H)
    wc1 = regroup(W_c1, F + H)

    full = lambda a: pl.BlockSpec(a.shape, lambda t: (0,) * a.ndim)
    args = (seq, idx, supA, supB,
            wg0, b_g0.reshape(1, -1), wc0, b_c0.reshape(1, -1),
            wg1, b_g1.reshape(1, -1), wc1, b_c1.reshape(1, -1),
            W_fc, b_fc.reshape(1, -1), W_id, b_id.reshape(1, -1))
    in_specs = [
        pl.BlockSpec((1, N, B * F), lambda t: (jnp.minimum(t, T - 1), 0, 0)),
        pl.BlockSpec(memory_space=pltpu.SMEM),
    ] + [full(a) for a in args[2:]]

    out_seiz, out_id = pl.pallas_call(
        _dcrnn_body,
        grid=(T + 1,),
        in_specs=in_specs,
        out_specs=[
            pl.BlockSpec((B, C_SEIZ), lambda t: (0, 0)),
            pl.BlockSpec((B, C_ID), lambda t: (0, 0)),
        ],
        out_shape=[
            jax.ShapeDtypeStruct((B, C_SEIZ), jnp.float32),
            jax.ShapeDtypeStruct((B, C_ID), jnp.float32),
        ],
        scratch_shapes=[
            pltpu.VMEM((N, B * H), jnp.float32),
            pltpu.VMEM((N, B * H), jnp.float32),
            pltpu.VMEM((N, B * H), jnp.bfloat16),
            pltpu.VMEM((B, N, H), jnp.float32),
        ],
        compiler_params=pltpu.CompilerParams(
            dimension_semantics=("arbitrary",),
        ),
    )(*args)
    return (out_seiz, out_id)
